# baseline (speedup 1.0000x reference)
import functools

import jax
import jax.numpy as jnp
from jax.experimental import pallas as pl
from jax.experimental.pallas import tpu as pltpu

_B = 2_000_000
_LANES = 128
_BR = 2048
_CH = _BR * _LANES
_G = -(-_B // _CH)


def _softplus(x):
  return jnp.maximum(x, 0.0) + jnp.log1p(jnp.exp(-jnp.abs(x)))


def _assoc(Dp, rho_a, rho_d):
  za = rho_a == 0.0
  zd = rho_d == 0.0
  a = jnp.where(za, 1.0, Dp * rho_a)
  d = jnp.where(zd, 1.0, Dp * rho_d)
  amd = a - d
  s = jnp.sqrt(amd * amd + 2.0 * (a + d) + 1.0)
  num_a = jnp.where(za, 1.0, a - 2.0 * d + 2.0)
  den_a = jnp.where(za, d, 2.0 * (s + 1.0 + d))
  Xa = num_a / den_a + jnp.where(za, 0.0, 0.5)
  Xa = jnp.where(zd, 1.0, Xa)
  num_d = jnp.where(zd, 1.0, d - 2.0 * a + 2.0)
  den_d = jnp.where(zd, a, 2.0 * (s + 1.0 + a))
  Xd = num_d / den_d + jnp.where(zd, 0.0, 0.5)
  Xd = jnp.where(za, 1.0, Xd)
  return Xa, Xd


def _body(dt_ref, dft_ref, mi_ref, i_ref, j_ref, invT_ref, r_ref,
          n_ref, rho_ref, o_ref):
  D = _softplus(dt_ref[...])
  Df = _softplus(dft_ref[...])
  m = mi_ref[...] != 0
  T = jnp.where(m, D, Df)

  two_d = lambda ref: ref[...].reshape(_BR, _LANES)
  ib = two_d(i_ref)
  jb = two_d(j_ref)
  shp = ib.shape
  t0 = jnp.broadcast_to(T[0:1, :], shp)
  t1 = jnp.broadcast_to(T[1:2, :], shp)
  gat = functools.partial(jnp.take_along_axis, axis=-1,
                          mode="promise_in_bounds")
  Ti0 = gat(t0, ib)
  Ti1 = gat(t1, ib)
  Tj0 = gat(t0, jb)
  Tj1 = gat(t1, jb)

  invT = two_d(invT_ref)
  dref = 0.034 * (jnp.exp(1960.0 * invT) - 1.0)
  D_AaAd = Ti0 * Ti1 * dref
  D_BaBd = Tj0 * Tj1 * dref
  D_AaBd = Ti0 * Tj1 * dref
  D_AdBa = Tj0 * Ti1 * dref

  comp4 = lambda c: rho_ref[c]
  comp2 = lambda c: n_ref[c]
  rBam = comp4(2)
  rBdm = comp4(3)
  XaBm, XdBm = _assoc(D_BaBd, rBam, rBdm)
  XaAp, XdAp = _assoc(D_AaAd, comp4(0), comp4(1))

  u_a = 1.0 + D_AaBd * rBdm * XdBm
  u_d = 1.0 + D_AdBa * rBam * XaBm
  naa = comp2(0)
  nad = comp2(1)
  termAa = jnp.where(naa == 0.0, 0.0,
                     naa * ((XaAp - 1.0) * 0.5 - jnp.log(u_a * XaAp)))
  termAd = jnp.where(nad == 0.0, 0.0,
                     nad * ((XdAp - 1.0) * 0.5 - jnp.log(u_d * XdAp)))
  termB = two_d(r_ref) * (rBam * (1.0 - XaBm) + rBdm * (1.0 - XdBm)) * 0.5
  o_ref[...] = (termAa + termAd + termB).reshape(_BR * _LANES)


def kernel(Delta, Delta_fixed, i, j, invT, r, q, N, rho, mask):
  del q
  dt = jnp.pad(Delta.T, ((0, 0), (0, _LANES - 64)))
  dft = jnp.pad(Delta_fixed.T, ((0, 0), (0, _LANES - 64)))
  mi = jnp.pad(mask[None, :].astype(jnp.int32), ((0, 0), (0, _LANES - 64)))

  blk = pl.BlockSpec((_CH,), lambda g: (g,))
  blk2 = pl.BlockSpec((2, _BR, _LANES), lambda g: (0, g, 0))
  blk4 = pl.BlockSpec((4, _BR, _LANES), lambda g: (0, g, 0))
  tab = pl.BlockSpec((2, _LANES), lambda g: (0, 0))
  mrow = pl.BlockSpec((1, _LANES), lambda g: (0, 0))

  _ROWS = _B // _LANES
  out = pl.pallas_call(
      _body,
      grid=(_G,),
      in_specs=[tab, tab, mrow] + [blk] * 4 + [blk2, blk4],
      out_specs=blk,
      out_shape=jax.ShapeDtypeStruct((_B,), jnp.float32),
  )(dt, dft, mi,
    i.astype(jnp.int32), j.astype(jnp.int32),
    invT, r,
    N.T.reshape(2, _ROWS, _LANES), rho.T.reshape(4, _ROWS, _LANES))
  return out


# speedup vs baseline: 1.0028x; 1.0028x over previous
import functools

import jax
import jax.numpy as jnp
from jax.experimental import pallas as pl

_B = 2_000_000
_LANES = 128
_BR = 2048
_CH = _BR * _LANES
_G = -(-_B // _CH)


def _softplus(x):
  return jnp.maximum(x, 0.0) + jnp.log1p(jnp.exp(-jnp.abs(x)))


def _assoc(Dp, rho_a, rho_d):
  za = rho_a == 0.0
  zd = rho_d == 0.0
  a = jnp.where(za, 1.0, Dp * rho_a)
  d = jnp.where(zd, 1.0, Dp * rho_d)
  amd = a - d
  s = jnp.sqrt(amd * amd + 2.0 * (a + d) + 1.0)
  num_a = jnp.where(za, 1.0, a - 2.0 * d + 2.0)
  den_a = jnp.where(za, d, 2.0 * (s + 1.0 + d))
  Xa = num_a / den_a + jnp.where(za, 0.0, 0.5)
  Xa = jnp.where(zd, 1.0, Xa)
  num_d = jnp.where(zd, 1.0, d - 2.0 * a + 2.0)
  den_d = jnp.where(zd, a, 2.0 * (s + 1.0 + a))
  Xd = num_d / den_d + jnp.where(zd, 0.0, 0.5)
  Xd = jnp.where(za, 1.0, Xd)
  return Xa, Xd


def _body(dt_ref, dft_ref, mi_ref, i_ref, j_ref, invT_ref, r_ref,
          n_ref, rho_ref, o_ref):
  D = _softplus(dt_ref[...])
  Df = _softplus(dft_ref[...])
  m = mi_ref[...] != 0
  T = jnp.where(m, D, Df)

  two_d = lambda ref: ref[...].reshape(_BR, _LANES)
  ib = two_d(i_ref)
  jb = two_d(j_ref)
  shp = ib.shape
  t0 = jnp.broadcast_to(T[0:1, :], shp)
  t1 = jnp.broadcast_to(T[1:2, :], shp)
  gat = functools.partial(jnp.take_along_axis, axis=-1,
                          mode="promise_in_bounds")
  Ti0 = gat(t0, ib)
  Ti1 = gat(t1, ib)
  Tj0 = gat(t0, jb)
  Tj1 = gat(t1, jb)

  invT = two_d(invT_ref)
  dref = 0.034 * (jnp.exp(1960.0 * invT) - 1.0)
  D_AaAd = Ti0 * Ti1 * dref
  D_BaBd = Tj0 * Tj1 * dref
  D_AaBd = Ti0 * Tj1 * dref
  D_AdBa = Tj0 * Ti1 * dref

  comp4 = lambda c: rho_ref[c]
  comp2 = lambda c: n_ref[c]
  rBam = comp4(2)
  rBdm = comp4(3)
  XaBm, XdBm = _assoc(D_BaBd, rBam, rBdm)
  XaAp, XdAp = _assoc(D_AaAd, comp4(0), comp4(1))

  u_a = 1.0 + D_AaBd * rBdm * XdBm
  u_d = 1.0 + D_AdBa * rBam * XaBm
  naa = comp2(0)
  nad = comp2(1)
  termAa = jnp.where(naa == 0.0, 0.0,
                     naa * ((XaAp - 1.0) * 0.5 - jnp.log(u_a * XaAp)))
  termAd = jnp.where(nad == 0.0, 0.0,
                     nad * ((XdAp - 1.0) * 0.5 - jnp.log(u_d * XdAp)))
  termB = two_d(r_ref) * (rBam * (1.0 - XaBm) + rBdm * (1.0 - XdBm)) * 0.5
  o_ref[...] = (termAa + termAd + termB).reshape(_BR * _LANES)


def kernel(Delta, Delta_fixed, i, j, invT, r, q, N, rho, mask):
  del q
  dt = jnp.pad(Delta.T, ((0, 0), (0, _LANES - 64)))
  dft = jnp.pad(Delta_fixed.T, ((0, 0), (0, _LANES - 64)))
  mi = jnp.pad(mask[None, :].astype(jnp.int32), ((0, 0), (0, _LANES - 64)))

  blk = pl.BlockSpec((_CH,), lambda g: (g,))
  blk2 = pl.BlockSpec((2, _BR, _LANES), lambda g: (0, g, 0))
  blk4 = pl.BlockSpec((4, _BR, _LANES), lambda g: (0, g, 0))
  tab = pl.BlockSpec((2, _LANES), lambda g: (0, 0))
  mrow = pl.BlockSpec((1, _LANES), lambda g: (0, 0))

  _ROWS = _B // _LANES
  out = pl.pallas_call(
      _body,
      grid=(_G,),
      in_specs=[tab, tab, mrow] + [blk] * 4 + [blk2, blk4],
      out_specs=blk,
      out_shape=jax.ShapeDtypeStruct((_B,), jnp.float32),
  )(dt, dft, mi,
    i.astype(jnp.int32), j.astype(jnp.int32),
    invT, r,
    N.T.reshape(2, _ROWS, _LANES), rho.T.reshape(4, _ROWS, _LANES))
  return out
